# baseline (speedup 1.0000x reference)
import functools

import jax
import jax.numpy as jnp
from jax import lax
from jax.experimental import pallas as pl
from jax.experimental.pallas import tpu as pltpu
from jax.experimental.pallas import tpu_sc as plsc

_BATCH = 16384
_LANES = 16
_NUM_WORKERS = 32
_B_PER_W = _BATCH // _NUM_WORKERS
_CHUNK = 32
_N_CHUNKS = _B_PER_W // _CHUNK
_GROUPS = _CHUNK // _LANES


def _sc_body(states_hbm, actions_hbm, tT_hbm, out_hbm,
             states_v, actions_v, buf0, buf1, out_v, sem0, sem1):
    wid = lax.axis_index("s") * 2 + lax.axis_index("c")
    base = wid * _B_PER_W

    pltpu.sync_copy(states_hbm.at[pl.ds(base, _B_PER_W)], states_v)
    pltpu.sync_copy(actions_hbm.at[pl.ds(base, _B_PER_W)], actions_v)

    bufs = [buf0, buf1]
    sems = [sem0, sem1]
    eidx = lax.iota(jnp.int32, _LANES)

    def fire(ci, buf, sem):
        for g in range(_GROUPS):
            s = states_v[pl.ds(ci * _CHUNK + g * _LANES, _LANES)]
            a = actions_v[pl.ds(ci * _CHUNK + g * _LANES, _LANES)]
            ab = lax.shift_right_logical(a, 3) * 8
            col = lax.shift_right_logical(s, 7) * 128
            for j in range(_LANES):
                abj = pl.multiple_of(ab[j], 8)
                colj = pl.multiple_of(col[j], 128)
                row = (g * _LANES + j) * 8
                pltpu.make_async_copy(
                    tT_hbm.at[pl.ds(abj, 8), pl.ds(colj, 128)],
                    buf.at[pl.ds(row, 8)], sem,
                ).start()

    def drain(buf, sem):
        for q in range(_CHUNK * 8 // 64):
            pltpu.make_async_copy(
                tT_hbm.at[pl.ds(0, 64), pl.ds(0, 128)],
                buf.at[pl.ds(q * 64, 64)], sem,
            ).wait()

    def extract(ci, buf):
        for g in range(_GROUPS):
            i = ci * _CHUNK + g * _LANES
            s = states_v[pl.ds(i, _LANES)]
            a = actions_v[pl.ds(i, _LANES)]
            row = (g * _LANES + eidx) * 8 + lax.bitwise_and(a, 7)
            lane = lax.bitwise_and(s, 127)
            x = plsc.load_gather(buf, [row, lane])
            out_v[pl.ds(i, _LANES)] = 1.0 / (1.0 + jnp.exp(-x))

    @pl.loop(-2, _N_CHUNKS, step=2)
    def _chunks(c):
        for b in range(2):
            ci = c + b

            @pl.when(ci >= 0)
            def _consume():
                drain(bufs[b], sems[b])
                extract(ci, bufs[b])

            @pl.when(ci + 2 < _N_CHUNKS)
            def _refill():
                fire(ci + 2, bufs[b], sems[b])

    pltpu.sync_copy(out_v, out_hbm.at[pl.ds(base, _B_PER_W)])


@functools.partial(jax.jit, static_argnames=())
def kernel(states, actions, rewards):
    tT = rewards.T
    mesh = plsc.VectorSubcoreMesh(core_axis_name="c", subcore_axis_name="s")
    run = pl.kernel(
        _sc_body,
        mesh=mesh,
        out_type=jax.ShapeDtypeStruct((_BATCH,), jnp.float32),
        scratch_types=[
            pltpu.VMEM((_B_PER_W,), jnp.int32),
            pltpu.VMEM((_B_PER_W,), jnp.int32),
            pltpu.VMEM((_CHUNK * 8, 128), jnp.float32),
            pltpu.VMEM((_CHUNK * 8, 128), jnp.float32),
            pltpu.VMEM((_B_PER_W,), jnp.float32),
            pltpu.SemaphoreType.DMA,
            pltpu.SemaphoreType.DMA,
        ],
        compiler_params=pltpu.CompilerParams(needs_layout_passes=False),
    )
    return run(states, actions, tT)


# speedup vs baseline: 1.0542x; 1.0542x over previous
import functools

import jax
import jax.numpy as jnp
from jax import lax
from jax.experimental import pallas as pl
from jax.experimental.pallas import tpu as pltpu
from jax.experimental.pallas import tpu_sc as plsc

_BATCH = 16384
_LANES = 16
_NUM_WORKERS = 32
_B_PER_W = _BATCH // _NUM_WORKERS
_CHUNK = 16
_NBUF = 4
_N_CHUNKS = _B_PER_W // _CHUNK
_GROUPS = _CHUNK // _LANES


def _sc_body(states_hbm, actions_hbm, tT_hbm, out_hbm,
             states_v, actions_v, buf0, buf1, buf2, buf3, out_v,
             sem0, sem1, sem2, sem3):
    wid = lax.axis_index("s") * 2 + lax.axis_index("c")
    base = wid * _B_PER_W

    pltpu.sync_copy(states_hbm.at[pl.ds(base, _B_PER_W)], states_v)
    pltpu.sync_copy(actions_hbm.at[pl.ds(base, _B_PER_W)], actions_v)

    bufs = [buf0, buf1, buf2, buf3]
    sems = [sem0, sem1, sem2, sem3]
    eidx = lax.iota(jnp.int32, _LANES)

    def fire(ci, buf, sem):
        s = states_v[pl.ds(ci * _CHUNK, _LANES)]
        a = actions_v[pl.ds(ci * _CHUNK, _LANES)]
        ab = lax.bitwise_and(a, ~7)
        col = lax.bitwise_and(s, ~127)
        for j in range(_LANES):
            abj = pl.multiple_of(ab[j], 8)
            colj = pl.multiple_of(col[j], 128)
            pltpu.make_async_copy(
                tT_hbm.at[pl.ds(abj, 8), pl.ds(colj, 128)],
                buf.at[pl.ds(j * 8, 8)], sem,
            ).start()

    def drain(buf, sem):
        for q in range(_CHUNK * 8 // 64):
            pltpu.make_async_copy(
                tT_hbm.at[pl.ds(0, 64), pl.ds(0, 128)],
                buf.at[pl.ds(q * 64, 64)], sem,
            ).wait()

    def extract(ci, buf):
        i = ci * _CHUNK
        s = states_v[pl.ds(i, _LANES)]
        a = actions_v[pl.ds(i, _LANES)]
        row = eidx * 8 + lax.bitwise_and(a, 7)
        lane = lax.bitwise_and(s, 127)
        x = plsc.load_gather(buf, [row, lane])
        out_v[pl.ds(i, _LANES)] = 1.0 / (1.0 + jnp.exp(-x))

    @pl.loop(-_NBUF, _N_CHUNKS, step=_NBUF)
    def _chunks(c):
        for b in range(_NBUF):
            ci = c + b

            @pl.when(ci >= 0)
            def _consume():
                drain(bufs[b], sems[b])
                extract(ci, bufs[b])

            @pl.when(ci + _NBUF < _N_CHUNKS)
            def _refill():
                fire(ci + _NBUF, bufs[b], sems[b])

    pltpu.sync_copy(out_v, out_hbm.at[pl.ds(base, _B_PER_W)])


@functools.partial(jax.jit, static_argnames=())
def kernel(states, actions, rewards):
    tT = rewards.T
    mesh = plsc.VectorSubcoreMesh(core_axis_name="c", subcore_axis_name="s")
    run = pl.kernel(
        _sc_body,
        mesh=mesh,
        out_type=jax.ShapeDtypeStruct((_BATCH,), jnp.float32),
        scratch_types=[
            pltpu.VMEM((_B_PER_W,), jnp.int32),
            pltpu.VMEM((_B_PER_W,), jnp.int32),
            pltpu.VMEM((_CHUNK * 8, 128), jnp.float32),
            pltpu.VMEM((_CHUNK * 8, 128), jnp.float32),
            pltpu.VMEM((_CHUNK * 8, 128), jnp.float32),
            pltpu.VMEM((_CHUNK * 8, 128), jnp.float32),
            pltpu.VMEM((_B_PER_W,), jnp.float32),
            pltpu.SemaphoreType.DMA,
            pltpu.SemaphoreType.DMA,
            pltpu.SemaphoreType.DMA,
            pltpu.SemaphoreType.DMA,
        ],
        compiler_params=pltpu.CompilerParams(needs_layout_passes=False),
    )
    return run(states, actions, tT)


# speedup vs baseline: 1.0931x; 1.0369x over previous
import functools

import jax
import jax.numpy as jnp
from jax import lax
from jax.experimental import pallas as pl
from jax.experimental.pallas import tpu as pltpu
from jax.experimental.pallas import tpu_sc as plsc

_BATCH = 16384
_LANES = 16
_NUM_WORKERS = 32
_B_PER_W = _BATCH // _NUM_WORKERS
_CHUNK = 16
_NBUF = 6
_N_CHUNKS = _B_PER_W // _CHUNK
_GROUPS = _CHUNK // _LANES


def _sc_body(states_hbm, actions_hbm, tT_hbm, out_hbm,
             states_v, actions_v, buf0, buf1, buf2, buf3, buf4, buf5,
             out_v, sem0, sem1, sem2, sem3, sem4, sem5):
    wid = lax.axis_index("s") * 2 + lax.axis_index("c")
    base = wid * _B_PER_W

    pltpu.sync_copy(states_hbm.at[pl.ds(base, _B_PER_W)], states_v)
    pltpu.sync_copy(actions_hbm.at[pl.ds(base, _B_PER_W)], actions_v)

    bufs = [buf0, buf1, buf2, buf3, buf4, buf5]
    sems = [sem0, sem1, sem2, sem3, sem4, sem5]
    eidx = lax.iota(jnp.int32, _LANES)

    def fire(ci, buf, sem):
        s = states_v[pl.ds(ci * _CHUNK, _LANES)]
        a = actions_v[pl.ds(ci * _CHUNK, _LANES)]
        ab = lax.bitwise_and(a, ~7)
        col = lax.bitwise_and(s, ~127)
        for j in range(_LANES):
            abj = pl.multiple_of(ab[j], 8)
            colj = pl.multiple_of(col[j], 128)
            pltpu.make_async_copy(
                tT_hbm.at[pl.ds(abj, 8), pl.ds(colj, 128)],
                buf.at[pl.ds(j * 8, 8)], sem,
            ).start()

    def drain(buf, sem):
        for q in range(_CHUNK * 8 // 64):
            pltpu.make_async_copy(
                tT_hbm.at[pl.ds(0, 64), pl.ds(0, 128)],
                buf.at[pl.ds(q * 64, 64)], sem,
            ).wait()

    def extract(ci, buf):
        i = ci * _CHUNK
        s = states_v[pl.ds(i, _LANES)]
        a = actions_v[pl.ds(i, _LANES)]
        row = eidx * 8 + lax.bitwise_and(a, 7)
        lane = lax.bitwise_and(s, 127)
        x = plsc.load_gather(buf, [row, lane])
        out_v[pl.ds(i, _LANES)] = 1.0 / (1.0 + jnp.exp(-x))

    @pl.loop(-_NBUF, _N_CHUNKS, step=_NBUF)
    def _chunks(c):
        for b in range(_NBUF):
            ci = c + b

            @pl.when((ci >= 0) & (ci < _N_CHUNKS))
            def _consume():
                drain(bufs[b], sems[b])
                extract(ci, bufs[b])

            @pl.when(ci + _NBUF < _N_CHUNKS)
            def _refill():
                fire(ci + _NBUF, bufs[b], sems[b])

    pltpu.sync_copy(out_v, out_hbm.at[pl.ds(base, _B_PER_W)])


@functools.partial(jax.jit, static_argnames=())
def kernel(states, actions, rewards):
    tT = rewards.T
    mesh = plsc.VectorSubcoreMesh(core_axis_name="c", subcore_axis_name="s")
    run = pl.kernel(
        _sc_body,
        mesh=mesh,
        out_type=jax.ShapeDtypeStruct((_BATCH,), jnp.float32),
        scratch_types=[
            pltpu.VMEM((_B_PER_W,), jnp.int32),
            pltpu.VMEM((_B_PER_W,), jnp.int32),
            pltpu.VMEM((_CHUNK * 8, 128), jnp.float32),
            pltpu.VMEM((_CHUNK * 8, 128), jnp.float32),
            pltpu.VMEM((_CHUNK * 8, 128), jnp.float32),
            pltpu.VMEM((_CHUNK * 8, 128), jnp.float32),
            pltpu.VMEM((_CHUNK * 8, 128), jnp.float32),
            pltpu.VMEM((_CHUNK * 8, 128), jnp.float32),
            pltpu.VMEM((_B_PER_W,), jnp.float32),
            pltpu.SemaphoreType.DMA,
            pltpu.SemaphoreType.DMA,
            pltpu.SemaphoreType.DMA,
            pltpu.SemaphoreType.DMA,
            pltpu.SemaphoreType.DMA,
            pltpu.SemaphoreType.DMA,
        ],
        compiler_params=pltpu.CompilerParams(needs_layout_passes=False),
    )
    return run(states, actions, tT)


# speedup vs baseline: 1.0950x; 1.0017x over previous
import functools

import jax
import jax.numpy as jnp
from jax import lax
from jax.experimental import pallas as pl
from jax.experimental.pallas import tpu as pltpu
from jax.experimental.pallas import tpu_sc as plsc

_BATCH = 16384
_LANES = 16
_NUM_WORKERS = 32
_B_PER_W = _BATCH // _NUM_WORKERS
_CHUNK = 16
_NBUF = 7
_N_CHUNKS = _B_PER_W // _CHUNK
_GROUPS = _CHUNK // _LANES


def _sc_body(states_hbm, actions_hbm, tT_hbm, out_hbm,
             states_v, actions_v, buf0, buf1, buf2, buf3, buf4, buf5,
             buf6, out_v, sem0, sem1, sem2, sem3, sem4, sem5, sem6):
    wid = lax.axis_index("s") * 2 + lax.axis_index("c")
    base = wid * _B_PER_W

    pltpu.sync_copy(states_hbm.at[pl.ds(base, _B_PER_W)], states_v)
    pltpu.sync_copy(actions_hbm.at[pl.ds(base, _B_PER_W)], actions_v)

    bufs = [buf0, buf1, buf2, buf3, buf4, buf5, buf6]
    sems = [sem0, sem1, sem2, sem3, sem4, sem5, sem6]
    eidx = lax.iota(jnp.int32, _LANES)

    def fire(ci, buf, sem):
        s = states_v[pl.ds(ci * _CHUNK, _LANES)]
        a = actions_v[pl.ds(ci * _CHUNK, _LANES)]
        ab = lax.bitwise_and(a, ~7)
        col = lax.bitwise_and(s, ~127)
        for j in range(_LANES):
            abj = pl.multiple_of(ab[j], 8)
            colj = pl.multiple_of(col[j], 128)
            pltpu.make_async_copy(
                tT_hbm.at[pl.ds(abj, 8), pl.ds(colj, 128)],
                buf.at[pl.ds(j * 8, 8)], sem,
            ).start()

    def drain(buf, sem):
        for q in range(_CHUNK * 8 // 64):
            pltpu.make_async_copy(
                tT_hbm.at[pl.ds(0, 64), pl.ds(0, 128)],
                buf.at[pl.ds(q * 64, 64)], sem,
            ).wait()

    def extract(ci, buf):
        i = ci * _CHUNK
        s = states_v[pl.ds(i, _LANES)]
        a = actions_v[pl.ds(i, _LANES)]
        row = eidx * 8 + lax.bitwise_and(a, 7)
        lane = lax.bitwise_and(s, 127)
        x = plsc.load_gather(buf, [row, lane])
        out_v[pl.ds(i, _LANES)] = 1.0 / (1.0 + jnp.exp(-x))

    @pl.loop(-_NBUF, _N_CHUNKS, step=_NBUF)
    def _chunks(c):
        for b in range(_NBUF):
            ci = c + b

            @pl.when((ci >= 0) & (ci < _N_CHUNKS))
            def _consume():
                drain(bufs[b], sems[b])
                extract(ci, bufs[b])

            @pl.when(ci + _NBUF < _N_CHUNKS)
            def _refill():
                fire(ci + _NBUF, bufs[b], sems[b])

    pltpu.sync_copy(out_v, out_hbm.at[pl.ds(base, _B_PER_W)])


@functools.partial(jax.jit, static_argnames=())
def kernel(states, actions, rewards):
    tT = rewards.T
    mesh = plsc.VectorSubcoreMesh(core_axis_name="c", subcore_axis_name="s")
    run = pl.kernel(
        _sc_body,
        mesh=mesh,
        out_type=jax.ShapeDtypeStruct((_BATCH,), jnp.float32),
        scratch_types=[
            pltpu.VMEM((_B_PER_W,), jnp.int32),
            pltpu.VMEM((_B_PER_W,), jnp.int32),
            pltpu.VMEM((_CHUNK * 8, 128), jnp.float32),
            pltpu.VMEM((_CHUNK * 8, 128), jnp.float32),
            pltpu.VMEM((_CHUNK * 8, 128), jnp.float32),
            pltpu.VMEM((_CHUNK * 8, 128), jnp.float32),
            pltpu.VMEM((_CHUNK * 8, 128), jnp.float32),
            pltpu.VMEM((_CHUNK * 8, 128), jnp.float32),
            pltpu.VMEM((_CHUNK * 8, 128), jnp.float32),
            pltpu.VMEM((_B_PER_W,), jnp.float32),
            pltpu.SemaphoreType.DMA,
            pltpu.SemaphoreType.DMA,
            pltpu.SemaphoreType.DMA,
            pltpu.SemaphoreType.DMA,
            pltpu.SemaphoreType.DMA,
            pltpu.SemaphoreType.DMA,
            pltpu.SemaphoreType.DMA,
        ],
        compiler_params=pltpu.CompilerParams(needs_layout_passes=False),
    )
    return run(states, actions, tT)
